# baseline (speedup 1.0000x reference)
import functools

import jax
import jax.numpy as jnp
from jax.experimental import pallas as pl
from jax.experimental.pallas import tpu as pltpu

_LN2 = 0.6931471805599453
_LOG2E = 1.4426950408889634


def _mlp_pool_kernel(x_ref, mask_ref, w1_ref, b1_ref, w2t_ref, b2_ref, y_ref):
    bB, A, D = x_ref.shape
    H = w1_ref.shape[1]
    x = x_ref[...].reshape(bB * A, D)
    a = jnp.dot(x, w1_ref[...], preferred_element_type=jnp.float32)
    a = a + b1_ref[...]
    t = jnp.exp2(a.astype(jnp.bfloat16)).astype(jnp.float32)
    r = jnp.log2(t + 0.5)
    m = mask_ref[...]
    hsum = jax.lax.dot_general(
        m, r.reshape(bB, A, H),
        dimension_numbers=(((1,), (1,)), ((0,), (0,))),
        preferred_element_type=jnp.float32,
    )
    msum = m.sum(axis=1, keepdims=True)
    y = jnp.sum(hsum * w2t_ref[...], axis=1, keepdims=True) + b2_ref[...] * msum
    y_ref[...] = y


@functools.partial(jax.jit, static_argnames=("block_b",))
def _run(representation, atom_mask, W1, b1, W2, b2, block_b=8):
    B, A, D = representation.shape
    H = W1.shape[1]
    w1s = W1 * _LOG2E
    b1s = ((b1 - _LN2) * _LOG2E).reshape(1, H)
    w2t = (W2 * _LN2).reshape(1, H)
    b2r = b2.reshape(1, 1)
    grid = (B // block_b,)
    y = pl.pallas_call(
        _mlp_pool_kernel,
        grid=grid,
        in_specs=[
            pl.BlockSpec((block_b, A, D), lambda i: (i, 0, 0)),
            pl.BlockSpec((block_b, A), lambda i: (i, 0)),
            pl.BlockSpec((D, H), lambda i: (0, 0)),
            pl.BlockSpec((1, H), lambda i: (0, 0)),
            pl.BlockSpec((1, H), lambda i: (0, 0)),
            pl.BlockSpec((1, 1), lambda i: (0, 0)),
        ],
        out_specs=pl.BlockSpec((block_b, 1), lambda i: (i, 0)),
        out_shape=jax.ShapeDtypeStruct((B, 1), jnp.float32),
        compiler_params=pltpu.CompilerParams(
            dimension_semantics=(pltpu.PARALLEL,),
        ),
    )(representation, atom_mask, w1s, b1s, w2t, b2r)
    return y


def kernel(representation, atomic_numbers, atom_mask, W1, b1, W2, b2):
    del atomic_numbers
    return _run(representation, atom_mask, W1, b1, W2, b2)


# speedup vs baseline: 1.0645x; 1.0645x over previous
import functools

import jax
import jax.numpy as jnp
from jax.experimental import pallas as pl
from jax.experimental.pallas import tpu as pltpu

_LN2 = 0.6931471805599453
_LOG2E = 1.4426950408889634


def _mlp_pool_kernel(x_ref, mask_ref, w1_ref, b1_ref, w2t_ref, b2_ref, y_ref):
    bB, A, D = x_ref.shape
    H = w1_ref.shape[1]
    w1s = w1_ref[...] * _LOG2E
    b1s = (b1_ref[...] - _LN2) * _LOG2E
    x = x_ref[...].reshape(bB * A, D)
    a = jnp.dot(x, w1s, preferred_element_type=jnp.float32) + b1s
    t = jnp.exp2(a.astype(jnp.bfloat16)).astype(jnp.float32)
    r = jnp.log2(t + 0.5)
    m = mask_ref[...]
    hsum = jax.lax.dot_general(
        m, r.reshape(bB, A, H),
        dimension_numbers=(((1,), (1,)), ((0,), (0,))),
        preferred_element_type=jnp.float32,
    )
    msum = m.sum(axis=1, keepdims=True)
    y = _LN2 * jnp.sum(hsum * w2t_ref[...], axis=1, keepdims=True)
    y_ref[...] = y + b2_ref[...] * msum


@functools.partial(jax.jit, static_argnames=("block_b",))
def _run(representation, atom_mask, W1, b1, W2, b2, block_b=8):
    B, A, D = representation.shape
    H = W1.shape[1]
    b1r = b1.reshape(1, H)
    w2t = W2.reshape(1, H)
    b2r = b2.reshape(1, 1)
    grid = (B // block_b,)
    y = pl.pallas_call(
        _mlp_pool_kernel,
        grid=grid,
        in_specs=[
            pl.BlockSpec((block_b, A, D), lambda i: (i, 0, 0)),
            pl.BlockSpec((block_b, A), lambda i: (i, 0)),
            pl.BlockSpec((D, H), lambda i: (0, 0)),
            pl.BlockSpec((1, H), lambda i: (0, 0)),
            pl.BlockSpec((1, H), lambda i: (0, 0)),
            pl.BlockSpec((1, 1), lambda i: (0, 0)),
        ],
        out_specs=pl.BlockSpec((block_b, 1), lambda i: (i, 0)),
        out_shape=jax.ShapeDtypeStruct((B, 1), jnp.float32),
        compiler_params=pltpu.CompilerParams(
            dimension_semantics=(pltpu.PARALLEL,),
        ),
    )(representation, atom_mask, W1, b1r, w2t, b2r)
    return y


def kernel(representation, atomic_numbers, atom_mask, W1, b1, W2, b2):
    del atomic_numbers
    return _run(representation, atom_mask, W1, b1, W2, b2)
